# Initial kernel scaffold; baseline (speedup 1.0000x reference)
import jax
import jax.numpy as jnp
from jax.experimental import pallas as pl


def kernel(x, prev_selected, w_router, ln1_g, ln1_b, Wq, Wk, Wv, Wo, ln2_g, ln2_b, W1, W2):
    raise NotImplementedError("write your pallas kernel here")


# speedup vs baseline: 725.5160x; 725.5160x over previous
import functools
import math

import jax
import jax.numpy as jnp
from jax import lax
from jax.experimental import pallas as pl
from jax.experimental.pallas import tpu as pltpu
from jax.experimental.pallas import tpu_sc as plsc

BB, TT, DD = 4, 2048, 2048
HH, DHD = 16, 128
DFF = 8192
TA = 1024
KS = 512

F32 = jnp.float32
BF16 = jnp.bfloat16


def _make_sc_gather(n_rows: int, chunk: int):
    mesh = plsc.VectorSubcoreMesh(core_axis_name="c", subcore_axis_name="s")
    info = plsc.get_sparse_core_info()
    nw = info.num_cores * info.num_subcores
    per_w = n_rows // nw
    n_chunks = per_w // chunk

    @functools.partial(
        pl.kernel,
        mesh=mesh,
        out_type=jax.ShapeDtypeStruct((n_rows, DD), F32),
        scratch_types=[
            pltpu.VMEM((per_w,), jnp.int32),
            pltpu.VMEM((chunk, DD), F32),
            pltpu.VMEM((chunk, DD), F32),
            pltpu.SemaphoreType.DMA,
            pltpu.SemaphoreType.DMA,
        ],
    )
    def gather_k(xf_hbm, idx_hbm, out_hbm, idx_v, buf0, buf1, sem0, sem1):
        wid = lax.axis_index("s") * info.num_cores + lax.axis_index("c")
        base = wid * per_w
        pltpu.sync_copy(idx_hbm.at[pl.ds(base, per_w)], idx_v)
        bufs = (buf0, buf1)
        sems = (sem0, sem1)
        copies = [None] * n_chunks
        copies[0] = pltpu.async_copy(
            xf_hbm.at[idx_v.at[pl.ds(0, chunk)]], bufs[0], sems[0])
        for c in range(n_chunks):
            if c + 1 < n_chunks:
                copies[c + 1] = pltpu.async_copy(
                    xf_hbm.at[idx_v.at[pl.ds((c + 1) * chunk, chunk)]],
                    bufs[(c + 1) % 2], sems[(c + 1) % 2])
            copies[c].wait()
            pltpu.sync_copy(bufs[c % 2],
                            out_hbm.at[pl.ds(base + c * chunk, chunk)])

    return gather_k


def _gather_rows(xf, idx_flat, n_rows):
    return _make_sc_gather(n_rows, 16)(xf, idx_flat)


_RB = 128


def _topk_body(logits_ref, ps_ref, sel_ref, gate_ref, aux_ref, z_ref,
               tv_acc, sel_acc, aux_acc, z_acc):
    b = pl.program_id(0)
    i = pl.program_id(1)
    vall = logits_ref[0, 0, :]
    vb = logits_ref[0, 0, pl.ds(i * _RB, _RB)]
    ps_blk = ps_ref[0, 0, pl.ds(i * _RB, _RB)]

    vi = vb.reshape(_RB, 1)
    vj = vall.reshape(1, TA)
    ii = i * _RB + lax.broadcasted_iota(jnp.int32, (_RB, TA), 0)
    jj = lax.broadcasted_iota(jnp.int32, (_RB, TA), 1)
    beats = (vj > vi) | ((vj == vi) & (jj < ii))
    rank = jnp.sum(beats.astype(F32), axis=1).astype(jnp.int32)

    kk = lax.broadcasted_iota(jnp.int32, (_RB, KS), 1)
    pm = rank.reshape(_RB, 1) == kk
    tv = jnp.sum(jnp.where(pm, vi, 0.0), axis=0)
    sv = jnp.sum(jnp.where(pm, ps_blk.reshape(_RB, 1), 0), axis=0)

    @pl.when(i == 0)
    def _init_b():
        tv_acc[...] = jnp.zeros((1, KS), F32)
        sel_acc[...] = jnp.zeros((1, KS), jnp.int32)

    @pl.when((b == 0) & (i == 0))
    def _init_all():
        aux_acc[0] = 0.0
        z_acc[0] = 0.0

    @pl.when(i == 0)
    def _zloss():
        vmax = jnp.max(vall)
        lse = vmax + jnp.log(jnp.sum(jnp.exp(vall - vmax)))
        z_acc[0] += lse * lse

    tv_acc[...] += tv.reshape(1, KS)
    sel_acc[...] += sv.reshape(1, KS)
    aux_acc[0] += jnp.sum(jax.nn.sigmoid(vb))

    sel_ref[...] = sel_acc[...].reshape(1, 1, KS)
    gate_ref[...] = jax.nn.sigmoid(tv_acc[...]).reshape(1, 1, KS)
    aux_ref[...] = jnp.full((1, 1), aux_acc[0] / (BB * TA), F32)
    z_ref[...] = jnp.full((1, 1), z_acc[0] / BB, F32)


def _router_topk(logits3, ps3):
    return pl.pallas_call(
        _topk_body,
        grid=(BB, TA // _RB),
        in_specs=[
            pl.BlockSpec((1, 1, TA), lambda b, i: (b, 0, 0)),
            pl.BlockSpec((1, 1, TA), lambda b, i: (b, 0, 0)),
        ],
        out_specs=[
            pl.BlockSpec((1, 1, KS), lambda b, i: (b, 0, 0)),
            pl.BlockSpec((1, 1, KS), lambda b, i: (b, 0, 0)),
            pl.BlockSpec((1, 1), lambda b, i: (0, 0)),
            pl.BlockSpec((1, 1), lambda b, i: (0, 0)),
        ],
        out_shape=[
            jax.ShapeDtypeStruct((BB, 1, KS), jnp.int32),
            jax.ShapeDtypeStruct((BB, 1, KS), F32),
            jax.ShapeDtypeStruct((1, 1), F32),
            jax.ShapeDtypeStruct((1, 1), F32),
        ],
        scratch_shapes=[
            pltpu.VMEM((1, KS), F32),
            pltpu.VMEM((1, KS), jnp.int32),
            pltpu.SMEM((1,), F32),
            pltpu.SMEM((1,), F32),
        ],
    )(logits3, ps3)


_CB = 512
_NCB = DD // _CB


def _qkv_body(x_ref, g_ref, b_ref, wq_ref, wk_ref, wv_ref,
              q_ref, k_ref, v_ref, a_scr):
    n = pl.program_id(1)

    @pl.when(n == 0)
    def _ln():
        xb = x_ref[0]
        mu = jnp.mean(xb, axis=1, keepdims=True)
        var = jnp.mean((xb - mu) ** 2, axis=1, keepdims=True)
        a = (xb - mu) * lax.rsqrt(var + 1e-5) * g_ref[...] + b_ref[...]
        a_scr[...] = a.astype(BF16)

    ab = a_scr[...]
    q_ref[0] = jnp.dot(ab, wq_ref[...], preferred_element_type=F32).astype(BF16)
    k_ref[0] = jnp.dot(ab, wk_ref[...], preferred_element_type=F32).astype(BF16)
    v_ref[0] = jnp.dot(ab, wv_ref[...], preferred_element_type=F32).astype(BF16)


def _qkv(xg3, ln1_g, ln1_b, wq, wk, wv):
    out_spec = pl.BlockSpec((1, KS, _CB), lambda b, n: (b, 0, n))
    return pl.pallas_call(
        _qkv_body,
        grid=(BB, _NCB),
        in_specs=[
            pl.BlockSpec((1, KS, DD), lambda b, n: (b, 0, 0)),
            pl.BlockSpec((1, DD), lambda b, n: (0, 0)),
            pl.BlockSpec((1, DD), lambda b, n: (0, 0)),
            pl.BlockSpec((DD, _CB), lambda b, n: (0, n)),
            pl.BlockSpec((DD, _CB), lambda b, n: (0, n)),
            pl.BlockSpec((DD, _CB), lambda b, n: (0, n)),
        ],
        out_specs=[out_spec, out_spec, out_spec],
        out_shape=[jax.ShapeDtypeStruct((BB, KS, DD), BF16)] * 3,
        scratch_shapes=[pltpu.VMEM((KS, DD), BF16)],
    )(xg3, ln1_g, ln1_b, wq, wk, wv)


def _attn_body(q_ref, k_ref, v_ref, o_ref):
    s = lax.dot_general(q_ref[0], k_ref[0], (((1,), (1,)), ((), ())),
                        preferred_element_type=F32)
    s = s * (1.0 / math.sqrt(DHD))
    ii = lax.broadcasted_iota(jnp.int32, (KS, KS), 0)
    jj = lax.broadcasted_iota(jnp.int32, (KS, KS), 1)
    s = jnp.where(jj <= ii, s, -1e9)
    m = jnp.max(s, axis=1, keepdims=True)
    e = jnp.exp(s - m)
    p = e / jnp.sum(e, axis=1, keepdims=True)
    o = jnp.dot(p.astype(BF16), v_ref[0], preferred_element_type=F32)
    o_ref[0] = o.astype(BF16)


def _attention(q, k, v):
    spec = pl.BlockSpec((1, KS, DHD), lambda b, h: (b, 0, h))
    return pl.pallas_call(
        _attn_body,
        grid=(BB, HH),
        in_specs=[spec, spec, spec],
        out_specs=spec,
        out_shape=jax.ShapeDtypeStruct((BB, KS, DD), BF16),
    )(q, k, v)


_NTOK = BB * KS
_TB1 = 256


def _postattn_body(x_ref, o_ref, wo_ref, g_ref, b_ref, h1_ref, m_ref):
    h1 = x_ref[...] + jnp.dot(o_ref[...], wo_ref[...],
                              preferred_element_type=F32)
    mu = jnp.mean(h1, axis=1, keepdims=True)
    var = jnp.mean((h1 - mu) ** 2, axis=1, keepdims=True)
    m = (h1 - mu) * lax.rsqrt(var + 1e-5) * g_ref[...] + b_ref[...]
    h1_ref[...] = h1.astype(BF16)
    m_ref[...] = m.astype(BF16)


def _postattn(xg2, o2, wo, ln2_g, ln2_b):
    nblk = _NTOK // _TB1
    return pl.pallas_call(
        _postattn_body,
        grid=(nblk,),
        in_specs=[
            pl.BlockSpec((_TB1, DD), lambda t: (t, 0)),
            pl.BlockSpec((_TB1, DD), lambda t: (t, 0)),
            pl.BlockSpec((DD, DD), lambda t: (0, 0)),
            pl.BlockSpec((1, DD), lambda t: (0, 0)),
            pl.BlockSpec((1, DD), lambda t: (0, 0)),
        ],
        out_specs=[
            pl.BlockSpec((_TB1, DD), lambda t: (t, 0)),
            pl.BlockSpec((_TB1, DD), lambda t: (t, 0)),
        ],
        out_shape=[
            jax.ShapeDtypeStruct((_NTOK, DD), BF16),
            jax.ShapeDtypeStruct((_NTOK, DD), BF16),
        ],
    )(xg2, o2, wo, ln2_g, ln2_b)


_TBM = 1024
_NTB = _NTOK // _TBM
_FB = 512
_NJ = DFF // _FB


def _mlp_body(m_ref, w1_ref, w2_ref, h1_ref, gate_ref, out_ref, acc_ref):
    j = pl.program_id(1)

    @pl.when(j == 0)
    def _init():
        acc_ref[...] = jnp.zeros((_TBM, DD), F32)

    f = jax.nn.gelu(jnp.dot(m_ref[...], w1_ref[...],
                            preferred_element_type=F32))
    acc_ref[...] += jnp.dot(f.astype(BF16), w2_ref[...],
                            preferred_element_type=F32)

    @pl.when(j == _NJ - 1)
    def _fin():
        h = h1_ref[...].astype(F32) + acc_ref[...]
        out_ref[...] = (h * gate_ref[...]).astype(BF16)


def _mlp(m2, w1, w2, h1, gate_col):
    return pl.pallas_call(
        _mlp_body,
        grid=(_NTB, _NJ),
        in_specs=[
            pl.BlockSpec((_TBM, DD), lambda t, j: (t, 0)),
            pl.BlockSpec((DD, _FB), lambda t, j: (0, j)),
            pl.BlockSpec((_FB, DD), lambda t, j: (j, 0)),
            pl.BlockSpec((_TBM, DD), lambda t, j: (t, 0)),
            pl.BlockSpec((_TBM, 1), lambda t, j: (t, 0)),
        ],
        out_specs=pl.BlockSpec((_TBM, DD), lambda t, j: (t, 0)),
        out_shape=jax.ShapeDtypeStruct((_NTOK, DD), BF16),
        scratch_shapes=[pltpu.VMEM((_TBM, DD), F32)],
    )(m2, w1, w2, h1, gate_col)


_TB2 = 256


def _scatter_body(x_ref, h_ref, sel_ref, out_ref):
    t = pl.program_id(1)
    rows = t * _TB2 + lax.broadcasted_iota(jnp.int32, (_TB2, KS), 0)
    s = (rows == sel_ref[0]).astype(BF16)
    delta = jnp.dot(s, h_ref[0], preferred_element_type=F32)
    out_ref[0] = x_ref[0] + delta


def _scatter(x, h_out, sel3):
    nblk = TT // _TB2
    return pl.pallas_call(
        _scatter_body,
        grid=(BB, nblk),
        in_specs=[
            pl.BlockSpec((1, _TB2, DD), lambda b, t: (b, t, 0)),
            pl.BlockSpec((1, KS, DD), lambda b, t: (b, 0, 0)),
            pl.BlockSpec((1, 1, KS), lambda b, t: (b, 0, 0)),
        ],
        out_specs=pl.BlockSpec((1, _TB2, DD), lambda b, t: (b, t, 0)),
        out_shape=jax.ShapeDtypeStruct((BB, TT, DD), F32),
    )(x, h_out, sel3)


def kernel(x, prev_selected, w_router, ln1_g, ln1_b, Wq, Wk, Wv, Wo,
           ln2_g, ln2_b, W1, W2):
    xf = x.reshape(BB * TT, DD)
    ps = prev_selected[..., 0]
    offs = (jnp.arange(BB, dtype=jnp.int32) * TT)[:, None]

    idx_a = (ps + offs).reshape(-1)
    active = _gather_rows(xf, idx_a, BB * TA).reshape(BB, TA, DD)

    logits3 = (active @ w_router)[..., 0].reshape(BB, 1, TA)
    sel3, gate3, aux, z = _router_topk(logits3, ps.reshape(BB, 1, TA))
    sel = sel3.reshape(BB, KS)

    idx_c = (sel + offs).reshape(-1)
    xg = _gather_rows(xf, idx_c, BB * KS)
    xg3 = xg.reshape(BB, KS, DD)

    q, k, v = _qkv(xg3, ln1_g.reshape(1, DD), ln1_b.reshape(1, DD),
                   Wq.astype(BF16), Wk.astype(BF16), Wv.astype(BF16))
    o = _attention(q, k, v)

    h1, m2 = _postattn(xg, o.reshape(BB * KS, DD), Wo.astype(BF16),
                       ln2_g.reshape(1, DD), ln2_b.reshape(1, DD))
    h_out = _mlp(m2, W1.astype(BF16), W2.astype(BF16), h1,
                 gate3.reshape(BB * KS, 1))

    total_x = _scatter(x, h_out.reshape(BB, KS, DD), sel3)

    return (total_x, sel.reshape(BB, KS, 1), aux.reshape(()),
            z.reshape(()), logits3.reshape(BB, TA))


# speedup vs baseline: 837.5925x; 1.1545x over previous
import functools
import math

import jax
import jax.numpy as jnp
from jax import lax
from jax.experimental import pallas as pl
from jax.experimental.pallas import tpu as pltpu
from jax.experimental.pallas import tpu_sc as plsc

BB, TT, DD = 4, 2048, 2048
HH, DHD = 16, 128
DFF = 8192
TA = 1024
KS = 512

F32 = jnp.float32
BF16 = jnp.bfloat16


def _make_sc_gather(n_rows: int, chunk: int):
    mesh = plsc.VectorSubcoreMesh(core_axis_name="c", subcore_axis_name="s")
    info = plsc.get_sparse_core_info()
    nw = info.num_cores * info.num_subcores
    per_w = n_rows // nw
    n_chunks = per_w // chunk

    @functools.partial(
        pl.kernel,
        mesh=mesh,
        out_type=jax.ShapeDtypeStruct((n_rows, DD), F32),
        scratch_types=[
            pltpu.VMEM((per_w,), jnp.int32),
            pltpu.VMEM((chunk, DD), F32),
            pltpu.VMEM((chunk, DD), F32),
            pltpu.SemaphoreType.DMA,
            pltpu.SemaphoreType.DMA,
        ],
    )
    def gather_k(xf_hbm, idx_hbm, out_hbm, idx_v, buf0, buf1, sem0, sem1):
        wid = lax.axis_index("s") * info.num_cores + lax.axis_index("c")
        base = wid * per_w
        pltpu.sync_copy(idx_hbm.at[pl.ds(base, per_w)], idx_v)
        bufs = (buf0, buf1)
        sems = (sem0, sem1)
        copies = [None] * n_chunks
        copies[0] = pltpu.async_copy(
            xf_hbm.at[idx_v.at[pl.ds(0, chunk)]], bufs[0], sems[0])
        for c in range(n_chunks):
            if c + 1 < n_chunks:
                copies[c + 1] = pltpu.async_copy(
                    xf_hbm.at[idx_v.at[pl.ds((c + 1) * chunk, chunk)]],
                    bufs[(c + 1) % 2], sems[(c + 1) % 2])
            copies[c].wait()
            pltpu.sync_copy(bufs[c % 2],
                            out_hbm.at[pl.ds(base + c * chunk, chunk)])

    return gather_k


def _gather_rows(xf, idx_flat, n_rows):
    return _make_sc_gather(n_rows, 16)(xf, idx_flat)


_RB = 128


def _topk_body(logits_ref, ps_ref, sel_ref, gate_ref, aux_ref, z_ref,
               tv_acc, sel_acc, aux_acc, z_acc):
    b = pl.program_id(0)
    i = pl.program_id(1)
    vall = logits_ref[0, 0, :]
    vb = logits_ref[0, 0, pl.ds(i * _RB, _RB)]
    ps_blk = ps_ref[0, 0, pl.ds(i * _RB, _RB)]

    vi = vb.reshape(_RB, 1)
    vj = vall.reshape(1, TA)
    ii = i * _RB + lax.broadcasted_iota(jnp.int32, (_RB, TA), 0)
    jj = lax.broadcasted_iota(jnp.int32, (_RB, TA), 1)
    beats = (vj > vi) | ((vj == vi) & (jj < ii))
    rank = jnp.sum(beats.astype(F32), axis=1).astype(jnp.int32)

    kk = lax.broadcasted_iota(jnp.int32, (_RB, KS), 1)
    pm = rank.reshape(_RB, 1) == kk
    tv = jnp.sum(jnp.where(pm, vi, 0.0), axis=0)
    sv = jnp.sum(jnp.where(pm, ps_blk.reshape(_RB, 1), 0), axis=0)

    @pl.when(i == 0)
    def _init_b():
        tv_acc[...] = jnp.zeros((1, KS), F32)
        sel_acc[...] = jnp.zeros((1, KS), jnp.int32)

    @pl.when((b == 0) & (i == 0))
    def _init_all():
        aux_acc[0] = 0.0
        z_acc[0] = 0.0

    @pl.when(i == 0)
    def _zloss():
        vmax = jnp.max(vall)
        lse = vmax + jnp.log(jnp.sum(jnp.exp(vall - vmax)))
        z_acc[0] += lse * lse

    tv_acc[...] += tv.reshape(1, KS)
    sel_acc[...] += sv.reshape(1, KS)
    aux_acc[0] += jnp.sum(jax.nn.sigmoid(vb))

    sel_ref[...] = sel_acc[...].reshape(1, 1, KS)
    gate_ref[...] = jax.nn.sigmoid(tv_acc[...]).reshape(1, 1, KS)
    aux_ref[...] = jnp.full((1, 1), aux_acc[0] / (BB * TA), F32)
    z_ref[...] = jnp.full((1, 1), z_acc[0] / BB, F32)


def _router_topk(logits3, ps3):
    return pl.pallas_call(
        _topk_body,
        grid=(BB, TA // _RB),
        in_specs=[
            pl.BlockSpec((1, 1, TA), lambda b, i: (b, 0, 0)),
            pl.BlockSpec((1, 1, TA), lambda b, i: (b, 0, 0)),
        ],
        out_specs=[
            pl.BlockSpec((1, 1, KS), lambda b, i: (b, 0, 0)),
            pl.BlockSpec((1, 1, KS), lambda b, i: (b, 0, 0)),
            pl.BlockSpec((1, 1), lambda b, i: (0, 0)),
            pl.BlockSpec((1, 1), lambda b, i: (0, 0)),
        ],
        out_shape=[
            jax.ShapeDtypeStruct((BB, 1, KS), jnp.int32),
            jax.ShapeDtypeStruct((BB, 1, KS), F32),
            jax.ShapeDtypeStruct((1, 1), F32),
            jax.ShapeDtypeStruct((1, 1), F32),
        ],
        scratch_shapes=[
            pltpu.VMEM((1, KS), F32),
            pltpu.VMEM((1, KS), jnp.int32),
            pltpu.SMEM((1,), F32),
            pltpu.SMEM((1,), F32),
        ],
    )(logits3, ps3)


_CB = 512
_NCB = DD // _CB


def _qkv_body(x_ref, g_ref, b_ref, wq_ref, wk_ref, wv_ref,
              q_ref, k_ref, v_ref, a_scr):
    n = pl.program_id(1)

    @pl.when(n == 0)
    def _ln():
        xb = x_ref[0]
        mu = jnp.mean(xb, axis=1, keepdims=True)
        var = jnp.mean((xb - mu) ** 2, axis=1, keepdims=True)
        a = (xb - mu) * lax.rsqrt(var + 1e-5) * g_ref[...] + b_ref[...]
        a_scr[...] = a.astype(BF16)

    ab = a_scr[...]
    q_ref[0] = jnp.dot(ab, wq_ref[...].astype(BF16),
                       preferred_element_type=F32).astype(BF16)
    k_ref[0] = jnp.dot(ab, wk_ref[...].astype(BF16),
                       preferred_element_type=F32).astype(BF16)
    v_ref[0] = jnp.dot(ab, wv_ref[...].astype(BF16),
                       preferred_element_type=F32).astype(BF16)


def _qkv(xg3, ln1_g, ln1_b, wq, wk, wv):
    out_spec = pl.BlockSpec((1, KS, _CB), lambda b, n: (b, 0, n))
    return pl.pallas_call(
        _qkv_body,
        grid=(BB, _NCB),
        in_specs=[
            pl.BlockSpec((1, KS, DD), lambda b, n: (b, 0, 0)),
            pl.BlockSpec((1, DD), lambda b, n: (0, 0)),
            pl.BlockSpec((1, DD), lambda b, n: (0, 0)),
            pl.BlockSpec((DD, _CB), lambda b, n: (0, n)),
            pl.BlockSpec((DD, _CB), lambda b, n: (0, n)),
            pl.BlockSpec((DD, _CB), lambda b, n: (0, n)),
        ],
        out_specs=[out_spec, out_spec, out_spec],
        out_shape=[jax.ShapeDtypeStruct((BB, KS, DD), BF16)] * 3,
        scratch_shapes=[pltpu.VMEM((KS, DD), BF16)],
    )(xg3, ln1_g, ln1_b, wq, wk, wv)


def _attn_body(q_ref, k_ref, v_ref, o_ref):
    s = lax.dot_general(q_ref[0], k_ref[0], (((1,), (1,)), ((), ())),
                        preferred_element_type=F32)
    s = s * (1.0 / math.sqrt(DHD))
    ii = lax.broadcasted_iota(jnp.int32, (KS, KS), 0)
    jj = lax.broadcasted_iota(jnp.int32, (KS, KS), 1)
    s = jnp.where(jj <= ii, s, -1e9)
    m = jnp.max(s, axis=1, keepdims=True)
    e = jnp.exp(s - m)
    p = e / jnp.sum(e, axis=1, keepdims=True)
    o = jnp.dot(p.astype(BF16), v_ref[0], preferred_element_type=F32)
    o_ref[0] = o.astype(BF16)


def _attention(q, k, v):
    spec = pl.BlockSpec((1, KS, DHD), lambda b, h: (b, 0, h))
    return pl.pallas_call(
        _attn_body,
        grid=(BB, HH),
        in_specs=[spec, spec, spec],
        out_specs=spec,
        out_shape=jax.ShapeDtypeStruct((BB, KS, DD), BF16),
    )(q, k, v)


_NTOK = BB * KS
_TB1 = 256


def _postattn_body(x_ref, o_ref, wo_ref, g_ref, b_ref, h1_ref, m_ref):
    h1 = x_ref[...] + jnp.dot(o_ref[...], wo_ref[...].astype(BF16),
                              preferred_element_type=F32)
    mu = jnp.mean(h1, axis=1, keepdims=True)
    var = jnp.mean((h1 - mu) ** 2, axis=1, keepdims=True)
    m = (h1 - mu) * lax.rsqrt(var + 1e-5) * g_ref[...] + b_ref[...]
    h1_ref[...] = h1.astype(BF16)
    m_ref[...] = m.astype(BF16)


def _postattn(xg2, o2, wo, ln2_g, ln2_b):
    nblk = _NTOK // _TB1
    return pl.pallas_call(
        _postattn_body,
        grid=(nblk,),
        in_specs=[
            pl.BlockSpec((_TB1, DD), lambda t: (t, 0)),
            pl.BlockSpec((_TB1, DD), lambda t: (t, 0)),
            pl.BlockSpec((DD, DD), lambda t: (0, 0)),
            pl.BlockSpec((1, DD), lambda t: (0, 0)),
            pl.BlockSpec((1, DD), lambda t: (0, 0)),
        ],
        out_specs=[
            pl.BlockSpec((_TB1, DD), lambda t: (t, 0)),
            pl.BlockSpec((_TB1, DD), lambda t: (t, 0)),
        ],
        out_shape=[
            jax.ShapeDtypeStruct((_NTOK, DD), BF16),
            jax.ShapeDtypeStruct((_NTOK, DD), BF16),
        ],
    )(xg2, o2, wo, ln2_g, ln2_b)


_TBM = 1024
_NTB = _NTOK // _TBM
_FB = 512
_NJ = DFF // _FB


def _mlp_body(m_ref, w1_ref, w2_ref, h1_ref, gate_ref, out_ref, acc_ref):
    j = pl.program_id(1)

    @pl.when(j == 0)
    def _init():
        acc_ref[...] = jnp.zeros((_TBM, DD), F32)

    f = jax.nn.gelu(jnp.dot(m_ref[...], w1_ref[...].astype(BF16),
                            preferred_element_type=F32))
    acc_ref[...] += jnp.dot(f.astype(BF16), w2_ref[...].astype(BF16),
                            preferred_element_type=F32)

    @pl.when(j == _NJ - 1)
    def _fin():
        h = h1_ref[...].astype(F32) + acc_ref[...]
        out_ref[...] = (h * gate_ref[...]).astype(BF16)


def _mlp(m2, w1, w2, h1, gate_col):
    return pl.pallas_call(
        _mlp_body,
        grid=(_NTB, _NJ),
        in_specs=[
            pl.BlockSpec((_TBM, DD), lambda t, j: (t, 0)),
            pl.BlockSpec((DD, _FB), lambda t, j: (0, j)),
            pl.BlockSpec((_FB, DD), lambda t, j: (j, 0)),
            pl.BlockSpec((_TBM, DD), lambda t, j: (t, 0)),
            pl.BlockSpec((_TBM, 1), lambda t, j: (t, 0)),
        ],
        out_specs=pl.BlockSpec((_TBM, DD), lambda t, j: (t, 0)),
        out_shape=jax.ShapeDtypeStruct((_NTOK, DD), BF16),
        scratch_shapes=[pltpu.VMEM((_TBM, DD), F32)],
    )(m2, w1, w2, h1, gate_col)


_TB2 = 256


def _scatter_body(x_ref, h_ref, sel_ref, out_ref):
    t = pl.program_id(1)
    rows = t * _TB2 + lax.broadcasted_iota(jnp.int32, (_TB2, KS), 0)
    s = (rows == sel_ref[0]).astype(BF16)
    delta = jnp.dot(s, h_ref[0], preferred_element_type=F32)
    out_ref[0] = x_ref[0] + delta


def _scatter(x, h_out, sel3):
    nblk = TT // _TB2
    return pl.pallas_call(
        _scatter_body,
        grid=(BB, nblk),
        in_specs=[
            pl.BlockSpec((1, _TB2, DD), lambda b, t: (b, t, 0)),
            pl.BlockSpec((1, KS, DD), lambda b, t: (b, 0, 0)),
            pl.BlockSpec((1, 1, KS), lambda b, t: (b, 0, 0)),
        ],
        out_specs=pl.BlockSpec((1, _TB2, DD), lambda b, t: (b, t, 0)),
        out_shape=jax.ShapeDtypeStruct((BB, TT, DD), F32),
    )(x, h_out, sel3)


def kernel(x, prev_selected, w_router, ln1_g, ln1_b, Wq, Wk, Wv, Wo,
           ln2_g, ln2_b, W1, W2):
    xf = x.reshape(BB * TT, DD)
    ps = prev_selected[..., 0]
    offs = (jnp.arange(BB, dtype=jnp.int32) * TT)[:, None]

    idx_a = (ps + offs).reshape(-1)
    active = _gather_rows(xf, idx_a, BB * TA).reshape(BB, TA, DD)

    logits3 = (active @ w_router)[..., 0].reshape(BB, 1, TA)
    sel3, gate3, aux, z = _router_topk(logits3, ps.reshape(BB, 1, TA))
    sel = sel3.reshape(BB, KS)

    idx_c = (sel + offs).reshape(-1)
    xg = _gather_rows(xf, idx_c, BB * KS)
    xg3 = xg.reshape(BB, KS, DD)

    q, k, v = _qkv(xg3, ln1_g.reshape(1, DD), ln1_b.reshape(1, DD),
                   Wq, Wk, Wv)
    o = _attention(q, k, v)

    h1, m2 = _postattn(xg, o.reshape(BB * KS, DD), Wo,
                       ln2_g.reshape(1, DD), ln2_b.reshape(1, DD))
    h_out = _mlp(m2, W1, W2, h1, gate3.reshape(BB * KS, 1))

    total_x = _scatter(x, h_out.reshape(BB, KS, DD), sel3)

    return (total_x, sel.reshape(BB, KS, 1), aux.reshape(()),
            z.reshape(()), logits3.reshape(BB, TA))


# speedup vs baseline: 925.4156x; 1.1049x over previous
import functools
import math

import jax
import jax.numpy as jnp
from jax import lax
from jax.experimental import pallas as pl
from jax.experimental.pallas import tpu as pltpu
from jax.experimental.pallas import tpu_sc as plsc

BB, TT, DD = 4, 2048, 2048
HH, DHD = 16, 128
DFF = 8192
TA = 1024
KS = 512

F32 = jnp.float32
BF16 = jnp.bfloat16


def _make_sc_gather(n_rows: int, chunk: int):
    mesh = plsc.VectorSubcoreMesh(core_axis_name="c", subcore_axis_name="s")
    info = plsc.get_sparse_core_info()
    nw = info.num_cores * info.num_subcores
    per_w = n_rows // nw
    n_chunks = per_w // chunk

    @functools.partial(
        pl.kernel,
        mesh=mesh,
        out_type=jax.ShapeDtypeStruct((n_rows, DD), F32),
        scratch_types=[
            pltpu.VMEM((per_w,), jnp.int32),
            pltpu.VMEM((chunk, DD), F32),
            pltpu.VMEM((chunk, DD), F32),
            pltpu.SemaphoreType.DMA,
            pltpu.SemaphoreType.DMA,
        ],
    )
    def gather_k(xf_hbm, idx_hbm, out_hbm, idx_v, buf0, buf1, sem0, sem1):
        wid = lax.axis_index("s") * info.num_cores + lax.axis_index("c")
        base = wid * per_w
        pltpu.sync_copy(idx_hbm.at[pl.ds(base, per_w)], idx_v)
        bufs = (buf0, buf1)
        sems = (sem0, sem1)
        copies = [None] * n_chunks
        copies[0] = pltpu.async_copy(
            xf_hbm.at[idx_v.at[pl.ds(0, chunk)]], bufs[0], sems[0])
        for c in range(n_chunks):
            if c + 1 < n_chunks:
                copies[c + 1] = pltpu.async_copy(
                    xf_hbm.at[idx_v.at[pl.ds((c + 1) * chunk, chunk)]],
                    bufs[(c + 1) % 2], sems[(c + 1) % 2])
            copies[c].wait()
            pltpu.sync_copy(bufs[c % 2],
                            out_hbm.at[pl.ds(base + c * chunk, chunk)])

    return gather_k


def _gather_rows(xf, idx_flat, n_rows):
    return _make_sc_gather(n_rows, 16)(xf, idx_flat)


_RB = 128


def _topk_body(logits_ref, ps_ref, sel_ref, gate_ref, aux_ref, z_ref,
               tv_acc, sel_acc, aux_acc, z_acc):
    b = pl.program_id(0)
    i = pl.program_id(1)
    vall = logits_ref[0, 0, :]
    vb = logits_ref[0, 0, pl.ds(i * _RB, _RB)]
    ps_blk = ps_ref[0, 0, pl.ds(i * _RB, _RB)]

    vi = vb.reshape(_RB, 1)
    vj = vall.reshape(1, TA)
    ii = i * _RB + lax.broadcasted_iota(jnp.int32, (_RB, TA), 0)
    jj = lax.broadcasted_iota(jnp.int32, (_RB, TA), 1)
    beats = (vj > vi) | ((vj == vi) & (jj < ii))
    rank = jnp.sum(beats.astype(F32), axis=1).astype(jnp.int32)

    kk = lax.broadcasted_iota(jnp.int32, (_RB, KS), 1)
    pm = rank.reshape(_RB, 1) == kk
    tv = jnp.sum(jnp.where(pm, vi, 0.0), axis=0)
    sv = jnp.sum(jnp.where(pm, ps_blk.reshape(_RB, 1), 0), axis=0)

    @pl.when(i == 0)
    def _init_b():
        tv_acc[...] = jnp.zeros((1, KS), F32)
        sel_acc[...] = jnp.zeros((1, KS), jnp.int32)

    @pl.when((b == 0) & (i == 0))
    def _init_all():
        aux_acc[0] = 0.0
        z_acc[0] = 0.0

    @pl.when(i == 0)
    def _zloss():
        vmax = jnp.max(vall)
        lse = vmax + jnp.log(jnp.sum(jnp.exp(vall - vmax)))
        z_acc[0] += lse * lse

    tv_acc[...] += tv.reshape(1, KS)
    sel_acc[...] += sv.reshape(1, KS)
    aux_acc[0] += jnp.sum(jax.nn.sigmoid(vb))

    sel_ref[...] = sel_acc[...].reshape(1, 1, KS)
    gate_ref[...] = jax.nn.sigmoid(tv_acc[...]).reshape(1, 1, KS)
    aux_ref[...] = jnp.full((1, 1), aux_acc[0] / (BB * TA), F32)
    z_ref[...] = jnp.full((1, 1), z_acc[0] / BB, F32)


def _router_topk(logits3, ps3):
    return pl.pallas_call(
        _topk_body,
        grid=(BB, TA // _RB),
        in_specs=[
            pl.BlockSpec((1, 1, TA), lambda b, i: (b, 0, 0)),
            pl.BlockSpec((1, 1, TA), lambda b, i: (b, 0, 0)),
        ],
        out_specs=[
            pl.BlockSpec((1, 1, KS), lambda b, i: (b, 0, 0)),
            pl.BlockSpec((1, 1, KS), lambda b, i: (b, 0, 0)),
            pl.BlockSpec((1, 1), lambda b, i: (0, 0)),
            pl.BlockSpec((1, 1), lambda b, i: (0, 0)),
        ],
        out_shape=[
            jax.ShapeDtypeStruct((BB, 1, KS), jnp.int32),
            jax.ShapeDtypeStruct((BB, 1, KS), F32),
            jax.ShapeDtypeStruct((1, 1), F32),
            jax.ShapeDtypeStruct((1, 1), F32),
        ],
        scratch_shapes=[
            pltpu.VMEM((1, KS), F32),
            pltpu.VMEM((1, KS), jnp.int32),
            pltpu.SMEM((1,), F32),
            pltpu.SMEM((1,), F32),
        ],
    )(logits3, ps3)


_CB = 512
_NCB = DD // _CB


def _qkv_attn_body(x_ref, g_ref, b_ref, wq_ref, wk_ref, wv_ref, o_ref,
                   a_scr, q_scr, k_scr, v_scr):
    n = pl.program_id(1)

    @pl.when(n == 0)
    def _ln():
        xb = x_ref[0]
        mu = jnp.mean(xb, axis=1, keepdims=True)
        var = jnp.mean((xb - mu) ** 2, axis=1, keepdims=True)
        a = (xb - mu) * lax.rsqrt(var + 1e-5) * g_ref[...] + b_ref[...]
        a_scr[...] = a.astype(BF16)

    ab = a_scr[...]
    cols = pl.ds(n * _CB, _CB)
    q_scr[:, cols] = jnp.dot(ab, wq_ref[...].astype(BF16),
                             preferred_element_type=F32).astype(BF16)
    k_scr[:, cols] = jnp.dot(ab, wk_ref[...].astype(BF16),
                             preferred_element_type=F32).astype(BF16)
    v_scr[:, cols] = jnp.dot(ab, wv_ref[...].astype(BF16),
                             preferred_element_type=F32).astype(BF16)

    @pl.when(n == _NCB - 1)
    def _attn():
        ii = lax.broadcasted_iota(jnp.int32, (KS, KS), 0)
        jj = lax.broadcasted_iota(jnp.int32, (KS, KS), 1)
        causal = jj <= ii
        for h in range(HH):
            hs = slice(h * DHD, (h + 1) * DHD)
            s = lax.dot_general(q_scr[:, hs], k_scr[:, hs],
                                (((1,), (1,)), ((), ())),
                                preferred_element_type=F32)
            s = s * (1.0 / math.sqrt(DHD))
            s = jnp.where(causal, s, -1e9)
            m = jnp.max(s, axis=1, keepdims=True)
            e = jnp.exp(s - m)
            p = e / jnp.sum(e, axis=1, keepdims=True)
            o = jnp.dot(p.astype(BF16), v_scr[:, hs],
                        preferred_element_type=F32)
            o_ref[0, :, hs] = o.astype(BF16)


def _qkv_attn(xg3, ln1_g, ln1_b, wq, wk, wv):
    return pl.pallas_call(
        _qkv_attn_body,
        grid=(BB, _NCB),
        in_specs=[
            pl.BlockSpec((1, KS, DD), lambda b, n: (b, 0, 0)),
            pl.BlockSpec((1, DD), lambda b, n: (0, 0)),
            pl.BlockSpec((1, DD), lambda b, n: (0, 0)),
            pl.BlockSpec((DD, _CB), lambda b, n: (0, n)),
            pl.BlockSpec((DD, _CB), lambda b, n: (0, n)),
            pl.BlockSpec((DD, _CB), lambda b, n: (0, n)),
        ],
        out_specs=pl.BlockSpec((1, KS, DD), lambda b, n: (b, 0, 0)),
        out_shape=jax.ShapeDtypeStruct((BB, KS, DD), BF16),
        scratch_shapes=[
            pltpu.VMEM((KS, DD), BF16),
            pltpu.VMEM((KS, DD), BF16),
            pltpu.VMEM((KS, DD), BF16),
            pltpu.VMEM((KS, DD), BF16),
        ],
    )(xg3, ln1_g, ln1_b, wq, wk, wv)


_NTOK = BB * KS
_TB1 = 256


def _postattn_body(x_ref, o_ref, wo_ref, g_ref, b_ref, h1_ref, m_ref):
    h1 = x_ref[...] + jnp.dot(o_ref[...], wo_ref[...].astype(BF16),
                              preferred_element_type=F32)
    mu = jnp.mean(h1, axis=1, keepdims=True)
    var = jnp.mean((h1 - mu) ** 2, axis=1, keepdims=True)
    m = (h1 - mu) * lax.rsqrt(var + 1e-5) * g_ref[...] + b_ref[...]
    h1_ref[...] = h1.astype(BF16)
    m_ref[...] = m.astype(BF16)


def _postattn(xg2, o2, wo, ln2_g, ln2_b):
    nblk = _NTOK // _TB1
    return pl.pallas_call(
        _postattn_body,
        grid=(nblk,),
        in_specs=[
            pl.BlockSpec((_TB1, DD), lambda t: (t, 0)),
            pl.BlockSpec((_TB1, DD), lambda t: (t, 0)),
            pl.BlockSpec((DD, DD), lambda t: (0, 0)),
            pl.BlockSpec((1, DD), lambda t: (0, 0)),
            pl.BlockSpec((1, DD), lambda t: (0, 0)),
        ],
        out_specs=[
            pl.BlockSpec((_TB1, DD), lambda t: (t, 0)),
            pl.BlockSpec((_TB1, DD), lambda t: (t, 0)),
        ],
        out_shape=[
            jax.ShapeDtypeStruct((_NTOK, DD), BF16),
            jax.ShapeDtypeStruct((_NTOK, DD), BF16),
        ],
    )(xg2, o2, wo, ln2_g, ln2_b)


_TBM = 1024
_NTB = _NTOK // _TBM
_FB = 512
_NJ = DFF // _FB


def _mlp_body(m_ref, w1_ref, w2_ref, h1_ref, gate_ref, out_ref, acc_ref):
    j = pl.program_id(1)

    @pl.when(j == 0)
    def _init():
        acc_ref[...] = jnp.zeros((_TBM, DD), F32)

    f = jax.nn.gelu(jnp.dot(m_ref[...], w1_ref[...].astype(BF16),
                            preferred_element_type=F32))
    acc_ref[...] += jnp.dot(f.astype(BF16), w2_ref[...].astype(BF16),
                            preferred_element_type=F32)

    @pl.when(j == _NJ - 1)
    def _fin():
        h = h1_ref[...].astype(F32) + acc_ref[...]
        out_ref[...] = (h * gate_ref[...]).astype(BF16)


def _mlp(m2, w1, w2, h1, gate_col):
    return pl.pallas_call(
        _mlp_body,
        grid=(_NTB, _NJ),
        in_specs=[
            pl.BlockSpec((_TBM, DD), lambda t, j: (t, 0)),
            pl.BlockSpec((DD, _FB), lambda t, j: (0, j)),
            pl.BlockSpec((_FB, DD), lambda t, j: (j, 0)),
            pl.BlockSpec((_TBM, DD), lambda t, j: (t, 0)),
            pl.BlockSpec((_TBM, 1), lambda t, j: (t, 0)),
        ],
        out_specs=pl.BlockSpec((_TBM, DD), lambda t, j: (t, 0)),
        out_shape=jax.ShapeDtypeStruct((_NTOK, DD), BF16),
        scratch_shapes=[pltpu.VMEM((_TBM, DD), F32)],
    )(m2, w1, w2, h1, gate_col)


_TB2 = 256


def _scatter_body(x_ref, h_ref, sel_ref, out_ref):
    t = pl.program_id(1)
    rows = t * _TB2 + lax.broadcasted_iota(jnp.int32, (_TB2, KS), 0)
    s = (rows == sel_ref[0]).astype(BF16)
    delta = jnp.dot(s, h_ref[0], preferred_element_type=F32)
    out_ref[0] = x_ref[0] + delta


def _scatter(x, h_out, sel3):
    nblk = TT // _TB2
    return pl.pallas_call(
        _scatter_body,
        grid=(BB, nblk),
        in_specs=[
            pl.BlockSpec((1, _TB2, DD), lambda b, t: (b, t, 0)),
            pl.BlockSpec((1, KS, DD), lambda b, t: (b, 0, 0)),
            pl.BlockSpec((1, 1, KS), lambda b, t: (b, 0, 0)),
        ],
        out_specs=pl.BlockSpec((1, _TB2, DD), lambda b, t: (b, t, 0)),
        out_shape=jax.ShapeDtypeStruct((BB, TT, DD), F32),
    )(x, h_out, sel3)


def kernel(x, prev_selected, w_router, ln1_g, ln1_b, Wq, Wk, Wv, Wo,
           ln2_g, ln2_b, W1, W2):
    xf = x.reshape(BB * TT, DD)
    ps = prev_selected[..., 0]
    offs = (jnp.arange(BB, dtype=jnp.int32) * TT)[:, None]

    idx_a = (ps + offs).reshape(-1)
    active = _gather_rows(xf, idx_a, BB * TA).reshape(BB, TA, DD)

    logits3 = (active @ w_router)[..., 0].reshape(BB, 1, TA)
    sel3, gate3, aux, z = _router_topk(logits3, ps.reshape(BB, 1, TA))
    sel = sel3.reshape(BB, KS)

    idx_c = (sel + offs).reshape(-1)
    xg = _gather_rows(xf, idx_c, BB * KS)
    xg3 = xg.reshape(BB, KS, DD)

    o = _qkv_attn(xg3, ln1_g.reshape(1, DD), ln1_b.reshape(1, DD),
                  Wq, Wk, Wv)

    h1, m2 = _postattn(xg, o.reshape(BB * KS, DD), Wo,
                       ln2_g.reshape(1, DD), ln2_b.reshape(1, DD))
    h_out = _mlp(m2, W1, W2, h1, gate3.reshape(BB * KS, 1))

    total_x = _scatter(x, h_out.reshape(BB, KS, DD), sel3)

    return (total_x, sel.reshape(BB, KS, 1), aux.reshape(()),
            z.reshape(()), logits3.reshape(BB, TA))


# speedup vs baseline: 933.8438x; 1.0091x over previous
import functools
import math

import jax
import jax.numpy as jnp
from jax import lax
from jax.experimental import pallas as pl
from jax.experimental.pallas import tpu as pltpu
from jax.experimental.pallas import tpu_sc as plsc

BB, TT, DD = 4, 2048, 2048
HH, DHD = 16, 128
DFF = 8192
TA = 1024
KS = 512

F32 = jnp.float32
BF16 = jnp.bfloat16


def _make_sc_gather(n_rows: int, chunk: int):
    mesh = plsc.VectorSubcoreMesh(core_axis_name="c", subcore_axis_name="s")
    info = plsc.get_sparse_core_info()
    nw = info.num_cores * info.num_subcores
    per_w = n_rows // nw
    n_chunks = per_w // chunk

    @functools.partial(
        pl.kernel,
        mesh=mesh,
        out_type=jax.ShapeDtypeStruct((n_rows, DD), F32),
        scratch_types=[
            pltpu.VMEM((per_w,), jnp.int32),
            pltpu.VMEM((chunk, DD), F32),
            pltpu.VMEM((chunk, DD), F32),
            pltpu.SemaphoreType.DMA,
            pltpu.SemaphoreType.DMA,
        ],
    )
    def gather_k(xf_hbm, idx_hbm, out_hbm, idx_v, buf0, buf1, sem0, sem1):
        wid = lax.axis_index("s") * info.num_cores + lax.axis_index("c")
        base = wid * per_w
        pltpu.sync_copy(idx_hbm.at[pl.ds(base, per_w)], idx_v)
        boff = (base // (n_rows // BB)) * TT
        for c in range(per_w // 16):
            sl = pl.ds(c * 16, 16)
            idx_v[sl] = idx_v[sl] + jnp.full((16,), boff, jnp.int32)
        bufs = (buf0, buf1)
        sems = (sem0, sem1)
        copies = [None] * n_chunks
        copies[0] = pltpu.async_copy(
            xf_hbm.at[idx_v.at[pl.ds(0, chunk)]], bufs[0], sems[0])
        for c in range(n_chunks):
            if c + 1 < n_chunks:
                copies[c + 1] = pltpu.async_copy(
                    xf_hbm.at[idx_v.at[pl.ds((c + 1) * chunk, chunk)]],
                    bufs[(c + 1) % 2], sems[(c + 1) % 2])
            copies[c].wait()
            pltpu.sync_copy(bufs[c % 2],
                            out_hbm.at[pl.ds(base + c * chunk, chunk)])

    return gather_k


def _gather_rows(xf, idx_flat, n_rows):
    return _make_sc_gather(n_rows, 16)(xf, idx_flat)


_RB = 128


def _topk_body(logits_ref, ps_ref, sel_ref, gate_ref, aux_ref, z_ref,
               tv_acc, sel_acc, aux_acc, z_acc):
    b = pl.program_id(0)
    i = pl.program_id(1)
    vall = logits_ref[0, 0, :]
    vb = logits_ref[0, 0, pl.ds(i * _RB, _RB)]
    ps_blk = ps_ref[0, 0, pl.ds(i * _RB, _RB)]

    vi = vb.reshape(_RB, 1)
    vj = vall.reshape(1, TA)
    ii = i * _RB + lax.broadcasted_iota(jnp.int32, (_RB, TA), 0)
    jj = lax.broadcasted_iota(jnp.int32, (_RB, TA), 1)
    beats = (vj > vi) | ((vj == vi) & (jj < ii))
    rank = jnp.sum(beats.astype(F32), axis=1).astype(jnp.int32)

    kk = lax.broadcasted_iota(jnp.int32, (_RB, KS), 1)
    pm = rank.reshape(_RB, 1) == kk
    tv = jnp.sum(jnp.where(pm, vi, 0.0), axis=0)
    sv = jnp.sum(jnp.where(pm, ps_blk.reshape(_RB, 1), 0), axis=0)

    @pl.when(i == 0)
    def _init_b():
        tv_acc[...] = jnp.zeros((1, KS), F32)
        sel_acc[...] = jnp.zeros((1, KS), jnp.int32)

    @pl.when((b == 0) & (i == 0))
    def _init_all():
        aux_acc[0] = 0.0
        z_acc[0] = 0.0

    @pl.when(i == 0)
    def _zloss():
        vmax = jnp.max(vall)
        lse = vmax + jnp.log(jnp.sum(jnp.exp(vall - vmax)))
        z_acc[0] += lse * lse

    tv_acc[...] += tv.reshape(1, KS)
    sel_acc[...] += sv.reshape(1, KS)
    aux_acc[0] += jnp.sum(jax.nn.sigmoid(vb))

    sel_ref[...] = sel_acc[...].reshape(1, 1, KS)
    gate_ref[...] = jax.nn.sigmoid(tv_acc[...]).reshape(1, 1, KS)
    aux_ref[...] = jnp.full((1, 1), aux_acc[0] / (BB * TA), F32)
    z_ref[...] = jnp.full((1, 1), z_acc[0] / BB, F32)


def _router_topk(logits3, ps3):
    return pl.pallas_call(
        _topk_body,
        grid=(BB, TA // _RB),
        in_specs=[
            pl.BlockSpec((1, 1, TA), lambda b, i: (b, 0, 0)),
            pl.BlockSpec((1, 1, TA), lambda b, i: (b, 0, 0)),
        ],
        out_specs=[
            pl.BlockSpec((1, 1, KS), lambda b, i: (b, 0, 0)),
            pl.BlockSpec((1, 1, KS), lambda b, i: (b, 0, 0)),
            pl.BlockSpec((1, 1), lambda b, i: (0, 0)),
            pl.BlockSpec((1, 1), lambda b, i: (0, 0)),
        ],
        out_shape=[
            jax.ShapeDtypeStruct((BB, 1, KS), jnp.int32),
            jax.ShapeDtypeStruct((BB, 1, KS), F32),
            jax.ShapeDtypeStruct((1, 1), F32),
            jax.ShapeDtypeStruct((1, 1), F32),
        ],
        scratch_shapes=[
            pltpu.VMEM((1, KS), F32),
            pltpu.VMEM((1, KS), jnp.int32),
            pltpu.SMEM((1,), F32),
            pltpu.SMEM((1,), F32),
        ],
    )(logits3, ps3)


_CB = 512
_NCB = DD // _CB


def _qkv_attn_body(x_ref, g_ref, b_ref, wq_ref, wk_ref, wv_ref, o_ref,
                   a_scr, q_scr, k_scr, v_scr):
    n = pl.program_id(1)

    @pl.when(n == 0)
    def _ln():
        xb = x_ref[0]
        mu = jnp.mean(xb, axis=1, keepdims=True)
        var = jnp.mean((xb - mu) ** 2, axis=1, keepdims=True)
        a = (xb - mu) * lax.rsqrt(var + 1e-5) * g_ref[...] + b_ref[...]
        a_scr[...] = a.astype(BF16)

    ab = a_scr[...]
    cols = pl.ds(n * _CB, _CB)
    q_scr[:, cols] = jnp.dot(ab, wq_ref[...].astype(BF16),
                             preferred_element_type=F32).astype(BF16)
    k_scr[:, cols] = jnp.dot(ab, wk_ref[...].astype(BF16),
                             preferred_element_type=F32).astype(BF16)
    v_scr[:, cols] = jnp.dot(ab, wv_ref[...].astype(BF16),
                             preferred_element_type=F32).astype(BF16)

    @pl.when(n == _NCB - 1)
    def _attn():
        ii = lax.broadcasted_iota(jnp.int32, (KS, KS), 0)
        jj = lax.broadcasted_iota(jnp.int32, (KS, KS), 1)
        causal = jj <= ii
        for h in range(HH):
            hs = slice(h * DHD, (h + 1) * DHD)
            s = lax.dot_general(q_scr[:, hs], k_scr[:, hs],
                                (((1,), (1,)), ((), ())),
                                preferred_element_type=F32)
            s = s * (1.0 / math.sqrt(DHD))
            s = jnp.where(causal, s, -1e9)
            m = jnp.max(s, axis=1, keepdims=True)
            e = jnp.exp(s - m)
            p = e / jnp.sum(e, axis=1, keepdims=True)
            o = jnp.dot(p.astype(BF16), v_scr[:, hs],
                        preferred_element_type=F32)
            o_ref[0, :, hs] = o.astype(BF16)


def _qkv_attn(xg3, ln1_g, ln1_b, wq, wk, wv):
    return pl.pallas_call(
        _qkv_attn_body,
        grid=(BB, _NCB),
        in_specs=[
            pl.BlockSpec((1, KS, DD), lambda b, n: (b, 0, 0)),
            pl.BlockSpec((1, DD), lambda b, n: (0, 0)),
            pl.BlockSpec((1, DD), lambda b, n: (0, 0)),
            pl.BlockSpec((DD, _CB), lambda b, n: (0, n)),
            pl.BlockSpec((DD, _CB), lambda b, n: (0, n)),
            pl.BlockSpec((DD, _CB), lambda b, n: (0, n)),
        ],
        out_specs=pl.BlockSpec((1, KS, DD), lambda b, n: (b, 0, 0)),
        out_shape=jax.ShapeDtypeStruct((BB, KS, DD), BF16),
        scratch_shapes=[
            pltpu.VMEM((KS, DD), BF16),
            pltpu.VMEM((KS, DD), BF16),
            pltpu.VMEM((KS, DD), BF16),
            pltpu.VMEM((KS, DD), BF16),
        ],
    )(xg3, ln1_g, ln1_b, wq, wk, wv)


_NTOK = BB * KS
_TB1 = 256


def _postattn_body(x_ref, o_ref, wo_ref, g_ref, b_ref, h1_ref, m_ref):
    h1 = x_ref[...] + jnp.dot(o_ref[...], wo_ref[...].astype(BF16),
                              preferred_element_type=F32)
    mu = jnp.mean(h1, axis=1, keepdims=True)
    var = jnp.mean((h1 - mu) ** 2, axis=1, keepdims=True)
    m = (h1 - mu) * lax.rsqrt(var + 1e-5) * g_ref[...] + b_ref[...]
    h1_ref[...] = h1.astype(BF16)
    m_ref[...] = m.astype(BF16)


def _postattn(xg2, o2, wo, ln2_g, ln2_b):
    nblk = _NTOK // _TB1
    return pl.pallas_call(
        _postattn_body,
        grid=(nblk,),
        in_specs=[
            pl.BlockSpec((_TB1, DD), lambda t: (t, 0)),
            pl.BlockSpec((_TB1, DD), lambda t: (t, 0)),
            pl.BlockSpec((DD, DD), lambda t: (0, 0)),
            pl.BlockSpec((1, DD), lambda t: (0, 0)),
            pl.BlockSpec((1, DD), lambda t: (0, 0)),
        ],
        out_specs=[
            pl.BlockSpec((_TB1, DD), lambda t: (t, 0)),
            pl.BlockSpec((_TB1, DD), lambda t: (t, 0)),
        ],
        out_shape=[
            jax.ShapeDtypeStruct((_NTOK, DD), BF16),
            jax.ShapeDtypeStruct((_NTOK, DD), BF16),
        ],
    )(xg2, o2, wo, ln2_g, ln2_b)


_TBM = 1024
_NTB = _NTOK // _TBM
_FB = 512
_NJ = DFF // _FB


def _mlp_body(m_ref, w1_ref, w2_ref, h1_ref, gate_ref, out_ref, acc_ref):
    j = pl.program_id(1)

    @pl.when(j == 0)
    def _init():
        acc_ref[...] = jnp.zeros((_TBM, DD), F32)

    f = jax.nn.gelu(jnp.dot(m_ref[...], w1_ref[...].astype(BF16),
                            preferred_element_type=F32))
    acc_ref[...] += jnp.dot(f.astype(BF16), w2_ref[...].astype(BF16),
                            preferred_element_type=F32)

    @pl.when(j == _NJ - 1)
    def _fin():
        h = h1_ref[...].astype(F32) + acc_ref[...]
        out_ref[...] = (h * gate_ref[...]).astype(BF16)


def _mlp(m2, w1, w2, h1, gate_col):
    return pl.pallas_call(
        _mlp_body,
        grid=(_NTB, _NJ),
        in_specs=[
            pl.BlockSpec((_TBM, DD), lambda t, j: (t, 0)),
            pl.BlockSpec((DD, _FB), lambda t, j: (0, j)),
            pl.BlockSpec((_FB, DD), lambda t, j: (j, 0)),
            pl.BlockSpec((_TBM, DD), lambda t, j: (t, 0)),
            pl.BlockSpec((_TBM, 1), lambda t, j: (t, 0)),
        ],
        out_specs=pl.BlockSpec((_TBM, DD), lambda t, j: (t, 0)),
        out_shape=jax.ShapeDtypeStruct((_NTOK, DD), BF16),
        scratch_shapes=[pltpu.VMEM((_TBM, DD), F32)],
    )(m2, w1, w2, h1, gate_col)


_TB2 = 256


def _scatter_body(x_ref, h_ref, sel_ref, out_ref):
    t = pl.program_id(1)
    rows = t * _TB2 + lax.broadcasted_iota(jnp.int32, (_TB2, KS), 0)
    s = (rows == sel_ref[0]).astype(BF16)
    delta = jnp.dot(s, h_ref[0], preferred_element_type=F32)
    out_ref[0] = x_ref[0] + delta


def _scatter(x, h_out, sel3):
    nblk = TT // _TB2
    return pl.pallas_call(
        _scatter_body,
        grid=(BB, nblk),
        in_specs=[
            pl.BlockSpec((1, _TB2, DD), lambda b, t: (b, t, 0)),
            pl.BlockSpec((1, KS, DD), lambda b, t: (b, 0, 0)),
            pl.BlockSpec((1, 1, KS), lambda b, t: (b, 0, 0)),
        ],
        out_specs=pl.BlockSpec((1, _TB2, DD), lambda b, t: (b, t, 0)),
        out_shape=jax.ShapeDtypeStruct((BB, TT, DD), F32),
    )(x, h_out, sel3)


def kernel(x, prev_selected, w_router, ln1_g, ln1_b, Wq, Wk, Wv, Wo,
           ln2_g, ln2_b, W1, W2):
    xf = x.reshape(BB * TT, DD)
    ps = prev_selected[..., 0]

    active = _gather_rows(xf, ps.reshape(-1), BB * TA).reshape(BB, TA, DD)

    logits3 = (active @ w_router)[..., 0].reshape(BB, 1, TA)
    sel3, gate3, aux, z = _router_topk(logits3, ps.reshape(BB, 1, TA))
    sel = sel3.reshape(BB, KS)

    xg = _gather_rows(xf, sel.reshape(-1), BB * KS)
    xg3 = xg.reshape(BB, KS, DD)

    o = _qkv_attn(xg3, ln1_g.reshape(1, DD), ln1_b.reshape(1, DD),
                  Wq, Wk, Wv)

    h1, m2 = _postattn(xg, o.reshape(BB * KS, DD), Wo,
                       ln2_g.reshape(1, DD), ln2_b.reshape(1, DD))
    h_out = _mlp(m2, W1, W2, h1, gate3.reshape(BB * KS, 1))

    total_x = _scatter(x, h_out.reshape(BB, KS, DD), sel3)

    return (total_x, sel.reshape(BB, KS, 1), aux.reshape(()),
            z.reshape(()), logits3.reshape(BB, TA))
